# baseline (speedup 1.0000x reference)
import functools

import jax
import jax.numpy as jnp
from jax import lax
from jax.experimental import pallas as pl
from jax.experimental.pallas import tpu as pltpu
from jax.experimental.pallas import tpu_sc as plsc

_BATCH = 256
_SPLIT = 4
_ROWS_PER_STEP = 8
_NBUF = 2


@functools.cache
def _build(num_qrows_table, qelems):
    info = plsc.get_sparse_core_info()
    nc, ns = info.num_cores, info.num_subcores
    nw = nc * ns
    out_qrows = _BATCH * _SPLIT
    per_w = out_qrows // nw
    n_steps = per_w // _ROWS_PER_STEP
    mesh = plsc.VectorSubcoreMesh(core_axis_name="c", subcore_axis_name="s")

    @functools.partial(
        pl.kernel,
        mesh=mesh,
        out_type=jax.ShapeDtypeStruct((out_qrows, qelems), jnp.float32),
        scratch_types=[
            pltpu.VMEM((n_steps, _ROWS_PER_STEP), jnp.int32),
            pltpu.VMEM((_NBUF, _ROWS_PER_STEP, qelems), jnp.float32),
            pltpu.SemaphoreType.DMA,
            pltpu.SemaphoreType.DMA,
            pltpu.SemaphoreType.DMA,
            pltpu.SemaphoreType.DMA,
        ],
    )
    def gather(table_hbm, idx_hbm, out_hbm, idx_v, bufs, g0, g1, s0, s1):
        gsem = (g0, g1)
        ssem = (s0, s1)
        wid = lax.axis_index("s") * nc + lax.axis_index("c")
        base = wid * per_w
        pltpu.sync_copy(idx_hbm.at[wid], idx_v)

        def start_gather(step):
            slot = step % _NBUF
            return pltpu.async_copy(
                table_hbm.at[idx_v.at[step]], bufs.at[slot], gsem[slot])

        gathers = [start_gather(s) for s in range(_NBUF)]
        pending = []
        for step in range(n_steps):
            slot = step % _NBUF
            gathers[slot].wait()
            sc = pltpu.async_copy(
                bufs.at[slot],
                out_hbm.at[pl.ds(base + step * _ROWS_PER_STEP, _ROWS_PER_STEP)],
                ssem[slot])
            nxt = step + _NBUF
            if nxt < n_steps:
                sc.wait()
                gathers[slot] = start_gather(nxt)
            else:
                pending.append(sc)
        for sc in pending:
            sc.wait()

    return gather, nw, n_steps


def kernel(data_sub_trajectories, permutations, i):
    num_total, sub_len, c, w = data_sub_trajectories.shape
    row_elems = sub_len * c * w
    qelems = row_elems // _SPLIT
    mb_per_epoch = -(-num_total // _BATCH)

    i = jnp.asarray(i)
    epoch_i = i // mb_per_epoch
    batch_start = (i % mb_per_epoch) * _BATCH
    batch_idx = lax.dynamic_slice(
        permutations, (epoch_i, batch_start), (1, _BATCH))[0]

    gather, nw, n_steps = _build(num_total * _SPLIT, qelems)
    qidx = (batch_idx[:, None] * _SPLIT
            + jnp.arange(_SPLIT, dtype=jnp.int32)).reshape(
                nw, n_steps, _ROWS_PER_STEP)
    table = data_sub_trajectories.reshape(num_total * _SPLIT, qelems)
    out = gather(table, qidx)
    return out.reshape(_BATCH, sub_len, c, w)


# speedup vs baseline: 8.9564x; 8.9564x over previous
import functools

import jax
import jax.numpy as jnp
from jax import lax
from jax.experimental import pallas as pl
from jax.experimental.pallas import tpu as pltpu
from jax.experimental.pallas import tpu_sc as plsc

_BATCH = 256
_ROWS_PER_STEP = 2
_NBUF = 2


@functools.cache
def _build(table_shape):
    info = plsc.get_sparse_core_info()
    nc, ns = info.num_cores, info.num_subcores
    nw = nc * ns
    per_w = _BATCH // nw
    n_steps = per_w // _ROWS_PER_STEP
    row_shape = table_shape[1:]
    mesh = plsc.VectorSubcoreMesh(core_axis_name="c", subcore_axis_name="s")

    @functools.partial(
        pl.kernel,
        mesh=mesh,
        out_type=jax.ShapeDtypeStruct((_BATCH,) + row_shape, jnp.float32),
        scratch_types=[
            pltpu.VMEM((n_steps, _ROWS_PER_STEP), jnp.int32),
            pltpu.VMEM((_NBUF, _ROWS_PER_STEP) + row_shape, jnp.float32),
            pltpu.SemaphoreType.DMA,
            pltpu.SemaphoreType.DMA,
            pltpu.SemaphoreType.DMA,
            pltpu.SemaphoreType.DMA,
        ],
    )
    def gather(table_hbm, idx_hbm, out_hbm, idx_v, bufs, g0, g1, s0, s1):
        gsem = (g0, g1)
        ssem = (s0, s1)
        wid = lax.axis_index("s") * nc + lax.axis_index("c")
        base = wid * per_w
        pltpu.sync_copy(idx_hbm.at[wid], idx_v)

        def start_gather(step):
            slot = step % _NBUF
            return pltpu.async_copy(
                table_hbm.at[idx_v.at[step]], bufs.at[slot], gsem[slot])

        gathers = [start_gather(s) for s in range(_NBUF)]
        pending = []
        for step in range(n_steps):
            slot = step % _NBUF
            gathers[slot].wait()
            sc = pltpu.async_copy(
                bufs.at[slot],
                out_hbm.at[pl.ds(base + step * _ROWS_PER_STEP, _ROWS_PER_STEP)],
                ssem[slot])
            nxt = step + _NBUF
            if nxt < n_steps:
                sc.wait()
                gathers[slot] = start_gather(nxt)
            else:
                pending.append(sc)
        for sc in pending:
            sc.wait()

    return gather, nw, n_steps


def kernel(data_sub_trajectories, permutations, i):
    num_total = data_sub_trajectories.shape[0]
    mb_per_epoch = -(-num_total // _BATCH)

    i = jnp.asarray(i)
    epoch_i = i // mb_per_epoch
    batch_start = (i % mb_per_epoch) * _BATCH
    batch_idx = lax.dynamic_slice(
        permutations, (epoch_i, batch_start), (1, _BATCH))[0]

    gather, nw, n_steps = _build(data_sub_trajectories.shape)
    idx = batch_idx.reshape(nw, n_steps, _ROWS_PER_STEP)
    return gather(data_sub_trajectories, idx)


# speedup vs baseline: 9.0468x; 1.0101x over previous
import functools

import jax
import jax.numpy as jnp
from jax import lax
from jax.experimental import pallas as pl
from jax.experimental.pallas import tpu as pltpu
from jax.experimental.pallas import tpu_sc as plsc

_BATCH = 256
_ROWS_PER_STEP = 1
_NBUF = 4
_LOOKAHEAD = 2


@functools.cache
def _build(table_shape):
    info = plsc.get_sparse_core_info()
    nc, ns = info.num_cores, info.num_subcores
    nw = nc * ns
    per_w = _BATCH // nw
    n_steps = per_w // _ROWS_PER_STEP
    row_shape = table_shape[1:]
    mesh = plsc.VectorSubcoreMesh(core_axis_name="c", subcore_axis_name="s")

    @functools.partial(
        pl.kernel,
        mesh=mesh,
        out_type=jax.ShapeDtypeStruct((_BATCH,) + row_shape, jnp.float32),
        scratch_types=[
            pltpu.VMEM((n_steps, _ROWS_PER_STEP), jnp.int32),
            pltpu.VMEM((_NBUF, _ROWS_PER_STEP) + row_shape, jnp.float32),
        ] + [pltpu.SemaphoreType.DMA] * (2 * _NBUF),
    )
    def gather(table_hbm, idx_hbm, out_hbm, idx_v, bufs, *sems):
        gsem = sems[:_NBUF]
        ssem = sems[_NBUF:]
        wid = lax.axis_index("s") * nc + lax.axis_index("c")
        base = wid * per_w
        pltpu.sync_copy(idx_hbm.at[wid], idx_v)

        def start_gather(step):
            slot = step % _NBUF
            return pltpu.async_copy(
                table_hbm.at[idx_v.at[step]], bufs.at[slot], gsem[slot])

        gathers = [None] * _NBUF
        for s in range(min(_LOOKAHEAD, n_steps)):
            gathers[s % _NBUF] = start_gather(s)
        scatters = [None] * _NBUF
        for step in range(n_steps):
            slot = step % _NBUF
            gathers[slot].wait()
            scatters[slot] = pltpu.async_copy(
                bufs.at[slot],
                out_hbm.at[pl.ds(base + step * _ROWS_PER_STEP, _ROWS_PER_STEP)],
                ssem[slot])
            la = step + _LOOKAHEAD
            if la < n_steps:
                laslot = la % _NBUF
                if scatters[laslot] is not None:
                    scatters[laslot].wait()
                gathers[laslot] = start_gather(la)
        for sc in scatters:
            if sc is not None:
                sc.wait()

    return gather, nw, n_steps


def kernel(data_sub_trajectories, permutations, i):
    num_total = data_sub_trajectories.shape[0]
    mb_per_epoch = -(-num_total // _BATCH)

    i = jnp.asarray(i)
    epoch_i = i // mb_per_epoch
    batch_start = (i % mb_per_epoch) * _BATCH
    batch_idx = lax.dynamic_slice(
        permutations, (epoch_i, batch_start), (1, _BATCH))[0]

    gather, nw, n_steps = _build(data_sub_trajectories.shape)
    idx = batch_idx.reshape(nw, n_steps, _ROWS_PER_STEP)
    return gather(data_sub_trajectories, idx)


# speedup vs baseline: 9.3308x; 1.0314x over previous
import functools

import jax
import jax.numpy as jnp
from jax import lax
from jax.experimental import pallas as pl
from jax.experimental.pallas import tpu as pltpu
from jax.experimental.pallas import tpu_sc as plsc

_BATCH = 256
_ROWS_PER_STEP = 1
_NBUF = 5
_LOOKAHEAD = 3


@functools.cache
def _build(table_shape):
    info = plsc.get_sparse_core_info()
    nc, ns = info.num_cores, info.num_subcores
    nw = nc * ns
    per_w = _BATCH // nw
    n_steps = per_w // _ROWS_PER_STEP
    row_shape = table_shape[1:]
    mesh = plsc.VectorSubcoreMesh(core_axis_name="c", subcore_axis_name="s")

    @functools.partial(
        pl.kernel,
        mesh=mesh,
        out_type=jax.ShapeDtypeStruct((_BATCH,) + row_shape, jnp.float32),
        scratch_types=[
            pltpu.VMEM((n_steps, _ROWS_PER_STEP), jnp.int32),
            pltpu.VMEM((_NBUF, _ROWS_PER_STEP) + row_shape, jnp.float32),
        ] + [pltpu.SemaphoreType.DMA] * (2 * _NBUF),
    )
    def gather(table_hbm, idx_hbm, out_hbm, idx_v, bufs, *sems):
        gsem = sems[:_NBUF]
        ssem = sems[_NBUF:]
        wid = lax.axis_index("s") * nc + lax.axis_index("c")
        base = wid * per_w
        pltpu.sync_copy(idx_hbm.at[wid], idx_v)

        def start_gather(step):
            slot = step % _NBUF
            return pltpu.async_copy(
                table_hbm.at[idx_v.at[step]], bufs.at[slot], gsem[slot])

        gathers = [None] * _NBUF
        for s in range(min(_LOOKAHEAD, n_steps)):
            gathers[s % _NBUF] = start_gather(s)
        scatters = [None] * _NBUF
        for step in range(n_steps):
            slot = step % _NBUF
            gathers[slot].wait()
            scatters[slot] = pltpu.async_copy(
                bufs.at[slot],
                out_hbm.at[pl.ds(base + step * _ROWS_PER_STEP, _ROWS_PER_STEP)],
                ssem[slot])
            la = step + _LOOKAHEAD
            if la < n_steps:
                laslot = la % _NBUF
                if scatters[laslot] is not None:
                    scatters[laslot].wait()
                gathers[laslot] = start_gather(la)
        for sc in scatters:
            if sc is not None:
                sc.wait()

    return gather, nw, n_steps


def kernel(data_sub_trajectories, permutations, i):
    num_total = data_sub_trajectories.shape[0]
    mb_per_epoch = -(-num_total // _BATCH)

    i = jnp.asarray(i)
    epoch_i = i // mb_per_epoch
    batch_start = (i % mb_per_epoch) * _BATCH
    batch_idx = lax.dynamic_slice(
        permutations, (epoch_i, batch_start), (1, _BATCH))[0]

    gather, nw, n_steps = _build(data_sub_trajectories.shape)
    idx = batch_idx.reshape(nw, n_steps, _ROWS_PER_STEP)
    return gather(data_sub_trajectories, idx)
